# baseline (speedup 1.0000x reference)
import dataclasses
import functools

import jax
import jax.numpy as jnp
from jax import lax
from jax.experimental import pallas as pl
from jax.experimental.pallas import tpu as pltpu
from jax.experimental.pallas import tpu_sc as plsc

_ROWS = 1000
_COLS = 16384
_SC_COLS = 4096
_TC_COLS = _COLS - _SC_COLS
_BLOCK_COLS = 2048
_GRID = _TC_COLS // _BLOCK_COLS

_NW = 32
_WCOLS = _SC_COLS // _NW
_RC = 200
_LANES = 16


def _tc_body(yt_ref, yp_ref, out_ref, acc_ref):
    i = pl.program_id(0)

    @pl.when(i == 0)
    def _init():
        acc_ref[0] = 0.0
        acc_ref[1] = 0.0

    yt = yt_ref[...]
    yp = yp_ref[...]
    mask = (yt != 0.0) & (yp != 0.0)
    d = yt - yp
    sq = jnp.where(mask, d * d, 0.0)
    acc_ref[0] += jnp.sum(sq)
    acc_ref[1] += jnp.sum(mask.astype(jnp.float32))

    @pl.when(i == _GRID - 1)
    def _fin():
        out_ref[0, 0] = acc_ref[0]
        out_ref[0, 1] = acc_ref[1]


def _sc_body(yt_hbm, yp_hbm, out_hbm, bt, bp, outv):
    wid = lax.axis_index("s") * 2 + lax.axis_index("c")
    col0 = _TC_COLS + wid * _WCOLS

    acc = jnp.zeros((_LANES,), jnp.float32)
    cnt = jnp.zeros((_LANES,), jnp.float32)
    for c in range(_ROWS // _RC):
        rows = pl.ds(c * _RC, _RC)
        cols = pl.ds(col0, _WCOLS)
        pltpu.sync_copy(yt_hbm.at[rows, cols], bt)
        pltpu.sync_copy(yp_hbm.at[rows, cols], bp)

        def body(r, carry):
            a, n = carry
            for k in range(_WCOLS // _LANES):
                yt = bt[r, pl.ds(k * _LANES, _LANES)]
                yp = bp[r, pl.ds(k * _LANES, _LANES)]
                m = (yt != 0.0) & (yp != 0.0)
                d = yt - yp
                a = a + jnp.where(m, d * d, 0.0)
                n = n + jnp.where(m, 1.0, 0.0)
            return a, n

        acc, cnt = lax.fori_loop(0, _RC, body, (acc, cnt))

    outv[...] = jnp.full((_LANES,), jnp.sum(acc), jnp.float32)
    pltpu.sync_copy(outv, out_hbm.at[0, wid])
    outv[...] = jnp.full((_LANES,), jnp.sum(cnt), jnp.float32)
    pltpu.sync_copy(outv, out_hbm.at[1, wid])


_sc_cp = pltpu.CompilerParams()
if "needs_layout_passes" in pltpu.CompilerParams.__dataclass_fields__:
    _sc_cp = dataclasses.replace(_sc_cp, needs_layout_passes=False)

_sc_kernel = functools.partial(
    pl.kernel,
    mesh=plsc.VectorSubcoreMesh(core_axis_name="c", subcore_axis_name="s"),
    compiler_params=_sc_cp,
    out_type=jax.ShapeDtypeStruct((2, _NW, _LANES), jnp.float32),
    scratch_types=[
        pltpu.VMEM((_RC, _WCOLS), jnp.float32),
        pltpu.VMEM((_RC, _WCOLS), jnp.float32),
        pltpu.VMEM((_LANES,), jnp.float32),
    ],
)(_sc_body)


def kernel(y_true, y_pred):
    ytT = y_true.T
    ypT = y_pred.T

    sc_out = _sc_kernel(ytT, ypT)

    tc_out = pl.pallas_call(
        _tc_body,
        grid=(_GRID,),
        in_specs=[
            pl.BlockSpec((_ROWS, _BLOCK_COLS), lambda i: (0, i)),
            pl.BlockSpec((_ROWS, _BLOCK_COLS), lambda i: (0, i)),
        ],
        out_specs=pl.BlockSpec(
            (1, 2), lambda i: (0, 0), memory_space=pltpu.SMEM
        ),
        out_shape=jax.ShapeDtypeStruct((1, 2), jnp.float32),
        scratch_shapes=[pltpu.SMEM((2,), jnp.float32)],
    )(ytT, ypT)

    tot = tc_out[0, 0] + jnp.sum(sc_out[0, :, 0])
    cnt = tc_out[0, 1] + jnp.sum(sc_out[1, :, 0])
    return tot / cnt


# speedup vs baseline: 1.0991x; 1.0991x over previous
import dataclasses
import functools

import jax
import jax.numpy as jnp
from jax import lax
from jax.experimental import pallas as pl
from jax.experimental.pallas import tpu as pltpu
from jax.experimental.pallas import tpu_sc as plsc

_ROWS = 1000
_COLS = 16384
_SC_COLS = 4096
_TC_COLS = _COLS - _SC_COLS
_BLOCK_COLS = 2048
_GRID = _TC_COLS // _BLOCK_COLS

_NW = 32
_WCOLS = _SC_COLS // _NW
_RC = 200
_LANES = 16


def _tc_body(yt_ref, yp_ref, out_ref, acc_ref):
    i = pl.program_id(0)

    @pl.when(i == 0)
    def _init():
        acc_ref[0] = 0.0
        acc_ref[1] = 0.0

    yt = yt_ref[...]
    yp = yp_ref[...]
    mask = (yt != 0.0) & (yp != 0.0)
    d = yt - yp
    sq = jnp.where(mask, d * d, 0.0)
    acc_ref[0] += jnp.sum(sq)
    acc_ref[1] += jnp.sum(mask.astype(jnp.float32))

    @pl.when(i == _GRID - 1)
    def _fin():
        out_ref[0, 0] = acc_ref[0]
        out_ref[0, 1] = acc_ref[1]


def _sc_body(yt_hbm, yp_hbm, out_hbm, bt, bp, outv):
    wid = lax.axis_index("s") * 2 + lax.axis_index("c")
    col0 = _TC_COLS + wid * _WCOLS
    nk = _WCOLS // _LANES

    carry = (
        tuple(jnp.zeros((_LANES,), jnp.float32) for _ in range(nk)),
        tuple(jnp.zeros((_LANES,), jnp.int32) for _ in range(nk)),
    )
    for c in range(_ROWS // _RC):
        rows = pl.ds(c * _RC, _RC)
        cols = pl.ds(col0, _WCOLS)
        pltpu.sync_copy(yt_hbm.at[rows, cols], bt)
        pltpu.sync_copy(yp_hbm.at[rows, cols], bp)

        def body(r, kcarry):
            accs, cnts = kcarry
            new_accs, new_cnts = [], []
            for k in range(nk):
                yt = bt[r, pl.ds(k * _LANES, _LANES)]
                yp = bp[r, pl.ds(k * _LANES, _LANES)]
                m = (yt != 0.0) & (yp != 0.0)
                d = yt - yp
                new_accs.append(accs[k] + jnp.where(m, d * d, 0.0))
                new_cnts.append(cnts[k] + plsc.all_reduce_population_count(m))
            return tuple(new_accs), tuple(new_cnts)

        carry = plsc.parallel_loop(0, _RC, carry=carry)(body)

    accs, cnts = carry
    acc = accs[0]
    for k in range(1, nk):
        acc = acc + accs[k]
    cntv = cnts[0]
    for k in range(1, nk):
        cntv = cntv + cnts[k]
    cnt_f = jnp.max(cntv).astype(jnp.float32)
    outv[...] = jnp.full((_LANES,), jnp.sum(acc), jnp.float32)
    pltpu.sync_copy(outv, out_hbm.at[0, wid])
    outv[...] = jnp.full((_LANES,), cnt_f, jnp.float32)
    pltpu.sync_copy(outv, out_hbm.at[1, wid])


_sc_cp = pltpu.CompilerParams()
if "needs_layout_passes" in pltpu.CompilerParams.__dataclass_fields__:
    _sc_cp = dataclasses.replace(_sc_cp, needs_layout_passes=False)

_sc_kernel = functools.partial(
    pl.kernel,
    mesh=plsc.VectorSubcoreMesh(core_axis_name="c", subcore_axis_name="s"),
    compiler_params=_sc_cp,
    out_type=jax.ShapeDtypeStruct((2, _NW, _LANES), jnp.float32),
    scratch_types=[
        pltpu.VMEM((_RC, _WCOLS), jnp.float32),
        pltpu.VMEM((_RC, _WCOLS), jnp.float32),
        pltpu.VMEM((_LANES,), jnp.float32),
    ],
)(_sc_body)


def kernel(y_true, y_pred):
    ytT = y_true.T
    ypT = y_pred.T

    sc_out = _sc_kernel(ytT, ypT)

    tc_out = pl.pallas_call(
        _tc_body,
        grid=(_GRID,),
        in_specs=[
            pl.BlockSpec((_ROWS, _BLOCK_COLS), lambda i: (0, i)),
            pl.BlockSpec((_ROWS, _BLOCK_COLS), lambda i: (0, i)),
        ],
        out_specs=pl.BlockSpec(
            (1, 2), lambda i: (0, 0), memory_space=pltpu.SMEM
        ),
        out_shape=jax.ShapeDtypeStruct((1, 2), jnp.float32),
        scratch_shapes=[pltpu.SMEM((2,), jnp.float32)],
    )(ytT, ypT)

    tot = tc_out[0, 0] + jnp.sum(sc_out[0, :, 0])
    cnt = tc_out[0, 1] + jnp.sum(sc_out[1, :, 0])
    return tot / cnt


# speedup vs baseline: 1.8044x; 1.6418x over previous
import dataclasses
import functools

import jax
import jax.numpy as jnp
from jax import lax
from jax.experimental import pallas as pl
from jax.experimental.pallas import tpu as pltpu
from jax.experimental.pallas import tpu_sc as plsc

_ROWS = 1000
_COLS = 16384
_SC_COLS = 4096
_TC_COLS = _COLS - _SC_COLS
_BLOCK_COLS = 2048
_GRID = _TC_COLS // _BLOCK_COLS

_NW = 32
_WCOLS = _SC_COLS // _NW
_RC = 200
_LANES = 16


def _tc_body(yt_ref, yp_ref, out_ref, acc_ref):
    i = pl.program_id(0)

    @pl.when(i == 0)
    def _init():
        acc_ref[0] = 0.0
        acc_ref[1] = 0.0

    yt = yt_ref[...]
    yp = yp_ref[...]
    mask = (yt != 0.0) & (yp != 0.0)
    d = yt - yp
    sq = jnp.where(mask, d * d, 0.0)
    acc_ref[0] += jnp.sum(sq)
    acc_ref[1] += jnp.sum(mask.astype(jnp.float32))

    @pl.when(i == _GRID - 1)
    def _fin():
        out_ref[0, 0] = acc_ref[0]
        out_ref[0, 1] = acc_ref[1]


def _sc_body(yt_hbm, yp_hbm, out_hbm, bt, bp, outv):
    wid = lax.axis_index("s") * 2 + lax.axis_index("c")
    nk = 4
    nbands = _ROWS // 8
    max_b = -(-nbands // _NW)

    accs = tuple(jnp.zeros((_LANES,), jnp.float32) for _ in range(nk))
    cnts = tuple(jnp.zeros((_LANES,), jnp.int32) for _ in range(nk))
    cols = pl.ds(_TC_COLS, _SC_COLS)
    for b in range(max_b):
        bidx = wid + _NW * b
        valid = bidx < nbands
        row0 = pl.multiple_of(jnp.minimum(bidx, nbands - 1) * 8, 8)
        pltpu.sync_copy(yt_hbm.at[pl.ds(row0, 8), cols], bt)
        pltpu.sync_copy(yp_hbm.at[pl.ds(row0, 8), cols], bp)

        band = (
            tuple(jnp.zeros((_LANES,), jnp.float32) for _ in range(nk)),
            tuple(jnp.zeros((_LANES,), jnp.int32) for _ in range(nk)),
        )
        for r in range(8):
            def body(e, kcarry, r=r):
                a_t, n_t = kcarry
                na, nn = [], []
                for k in range(nk):
                    yt = bt[r, pl.ds(e + k * _LANES, _LANES)]
                    yp = bp[r, pl.ds(e + k * _LANES, _LANES)]
                    m = (yt != 0.0) & (yp != 0.0)
                    d = yt - yp
                    na.append(a_t[k] + jnp.where(m, d * d, 0.0))
                    nn.append(n_t[k] + plsc.all_reduce_population_count(m))
                return tuple(na), tuple(nn)

            band = plsc.parallel_loop(
                0, _SC_COLS, step=nk * _LANES, carry=band
            )(body)
        baccs, bcnts = band
        accs = tuple(
            a + jnp.where(valid, ba, 0.0) for a, ba in zip(accs, baccs)
        )
        cnts = tuple(n + jnp.where(valid, bn, 0) for n, bn in zip(cnts, bcnts))

    acc = accs[0]
    for k in range(1, nk):
        acc = acc + accs[k]
    cntv = cnts[0]
    for k in range(1, nk):
        cntv = cntv + cnts[k]
    cnt_f = jnp.max(cntv).astype(jnp.float32)
    outv[...] = jnp.full((_LANES,), jnp.sum(acc), jnp.float32)
    pltpu.sync_copy(outv, out_hbm.at[0, wid])
    outv[...] = jnp.full((_LANES,), cnt_f, jnp.float32)
    pltpu.sync_copy(outv, out_hbm.at[1, wid])


_sc_cp = pltpu.CompilerParams()
if "needs_layout_passes" in pltpu.CompilerParams.__dataclass_fields__:
    _sc_cp = dataclasses.replace(_sc_cp, needs_layout_passes=False)

_sc_kernel = functools.partial(
    pl.kernel,
    mesh=plsc.VectorSubcoreMesh(core_axis_name="c", subcore_axis_name="s"),
    compiler_params=_sc_cp,
    out_type=jax.ShapeDtypeStruct((2, _NW, _LANES), jnp.float32),
    scratch_types=[
        pltpu.VMEM((8, _SC_COLS), jnp.float32),
        pltpu.VMEM((8, _SC_COLS), jnp.float32),
        pltpu.VMEM((_LANES,), jnp.float32),
    ],
)(_sc_body)


def kernel(y_true, y_pred):
    ytT = y_true.T
    ypT = y_pred.T

    sc_out = _sc_kernel(ytT, ypT)

    tc_out = pl.pallas_call(
        _tc_body,
        grid=(_GRID,),
        in_specs=[
            pl.BlockSpec((_ROWS, _BLOCK_COLS), lambda i: (0, i)),
            pl.BlockSpec((_ROWS, _BLOCK_COLS), lambda i: (0, i)),
        ],
        out_specs=pl.BlockSpec(
            (1, 2), lambda i: (0, 0), memory_space=pltpu.SMEM
        ),
        out_shape=jax.ShapeDtypeStruct((1, 2), jnp.float32),
        scratch_shapes=[pltpu.SMEM((2,), jnp.float32)],
    )(ytT, ypT)

    tot = tc_out[0, 0] + jnp.sum(sc_out[0, :, 0])
    cnt = tc_out[0, 1] + jnp.sum(sc_out[1, :, 0])
    return tot / cnt


# speedup vs baseline: 2.0066x; 1.1121x over previous
import dataclasses
import functools

import jax
import jax.numpy as jnp
from jax import lax
from jax.experimental import pallas as pl
from jax.experimental.pallas import tpu as pltpu
from jax.experimental.pallas import tpu_sc as plsc

_ROWS = 1000
_COLS = 16384
_SC_COLS = 2048
_TC_COLS = _COLS - _SC_COLS
_BLOCK_COLS = 2048
_GRID = _TC_COLS // _BLOCK_COLS

_NW = 32
_WCOLS = _SC_COLS // _NW
_RC = 200
_LANES = 16


def _tc_body(yt_ref, yp_ref, out_ref, acc_ref):
    i = pl.program_id(0)

    @pl.when(i == 0)
    def _init():
        acc_ref[0] = 0.0
        acc_ref[1] = 0.0

    yt = yt_ref[...]
    yp = yp_ref[...]
    mask = (yt != 0.0) & (yp != 0.0)
    d = yt - yp
    sq = jnp.where(mask, d * d, 0.0)
    acc_ref[0] += jnp.sum(sq)
    acc_ref[1] += jnp.sum(mask.astype(jnp.float32))

    @pl.when(i == _GRID - 1)
    def _fin():
        out_ref[0, 0] = acc_ref[0]
        out_ref[0, 1] = acc_ref[1]


def _sc_body(yt_hbm, yp_hbm, out_hbm, bt, bp, outv):
    wid = lax.axis_index("s") * 2 + lax.axis_index("c")
    nk = 4
    nbands = _ROWS // 8
    max_b = -(-nbands // _NW)

    accs = tuple(jnp.zeros((_LANES,), jnp.float32) for _ in range(nk))
    cnts = tuple(jnp.zeros((_LANES,), jnp.int32) for _ in range(nk))
    cols = pl.ds(_TC_COLS, _SC_COLS)
    for b in range(max_b):
        bidx = wid + _NW * b
        valid = bidx < nbands
        row0 = pl.multiple_of(jnp.minimum(bidx, nbands - 1) * 8, 8)
        pltpu.sync_copy(yt_hbm.at[pl.ds(row0, 8), cols], bt)
        pltpu.sync_copy(yp_hbm.at[pl.ds(row0, 8), cols], bp)

        band = (
            tuple(jnp.zeros((_LANES,), jnp.float32) for _ in range(nk)),
            tuple(jnp.zeros((_LANES,), jnp.int32) for _ in range(nk)),
        )
        for r in range(8):
            def body(e, kcarry, r=r):
                a_t, n_t = kcarry
                na, nn = [], []
                for k in range(nk):
                    yt = bt[r, pl.ds(e + k * _LANES, _LANES)]
                    yp = bp[r, pl.ds(e + k * _LANES, _LANES)]
                    m = (yt != 0.0) & (yp != 0.0)
                    d = yt - yp
                    na.append(a_t[k] + jnp.where(m, d * d, 0.0))
                    nn.append(n_t[k] + plsc.all_reduce_population_count(m))
                return tuple(na), tuple(nn)

            band = plsc.parallel_loop(
                0, _SC_COLS, step=nk * _LANES, carry=band
            )(body)
        baccs, bcnts = band
        accs = tuple(
            a + jnp.where(valid, ba, 0.0) for a, ba in zip(accs, baccs)
        )
        cnts = tuple(n + jnp.where(valid, bn, 0) for n, bn in zip(cnts, bcnts))

    acc = accs[0]
    for k in range(1, nk):
        acc = acc + accs[k]
    cntv = cnts[0]
    for k in range(1, nk):
        cntv = cntv + cnts[k]
    cnt_f = jnp.max(cntv).astype(jnp.float32)
    outv[...] = jnp.full((_LANES,), jnp.sum(acc), jnp.float32)
    pltpu.sync_copy(outv, out_hbm.at[0, wid])
    outv[...] = jnp.full((_LANES,), cnt_f, jnp.float32)
    pltpu.sync_copy(outv, out_hbm.at[1, wid])


_sc_cp = pltpu.CompilerParams()
if "needs_layout_passes" in pltpu.CompilerParams.__dataclass_fields__:
    _sc_cp = dataclasses.replace(_sc_cp, needs_layout_passes=False)

_sc_kernel = functools.partial(
    pl.kernel,
    mesh=plsc.VectorSubcoreMesh(core_axis_name="c", subcore_axis_name="s"),
    compiler_params=_sc_cp,
    out_type=jax.ShapeDtypeStruct((2, _NW, _LANES), jnp.float32),
    scratch_types=[
        pltpu.VMEM((8, _SC_COLS), jnp.float32),
        pltpu.VMEM((8, _SC_COLS), jnp.float32),
        pltpu.VMEM((_LANES,), jnp.float32),
    ],
)(_sc_body)


def kernel(y_true, y_pred):
    ytT = y_true.T
    ypT = y_pred.T

    sc_out = _sc_kernel(ytT, ypT)

    tc_out = pl.pallas_call(
        _tc_body,
        grid=(_GRID,),
        in_specs=[
            pl.BlockSpec((_ROWS, _BLOCK_COLS), lambda i: (0, i)),
            pl.BlockSpec((_ROWS, _BLOCK_COLS), lambda i: (0, i)),
        ],
        out_specs=pl.BlockSpec(
            (1, 2), lambda i: (0, 0), memory_space=pltpu.SMEM
        ),
        out_shape=jax.ShapeDtypeStruct((1, 2), jnp.float32),
        scratch_shapes=[pltpu.SMEM((2,), jnp.float32)],
    )(ytT, ypT)

    tot = tc_out[0, 0] + jnp.sum(sc_out[0, :, 0])
    cnt = tc_out[0, 1] + jnp.sum(sc_out[1, :, 0])
    return tot / cnt
